# baseline (speedup 1.0000x reference)
import functools

import jax
import jax.numpy as jnp
from jax import lax
from jax.experimental import pallas as pl
from jax.experimental.pallas import tpu as pltpu
from jax.experimental.pallas import tpu_sc as plsc

_N = 10000
_E = 320000
_F = 128
_FE = 16
_H = 32
_MSG = 4

_NW = 32
_EW = _E // _NW
_GW = 80
_FOLD = 4
_E4 = _E // _FOLD
_CHUNK = 1000


def _tc_precompute(x, w1a, w1b):
    def body(x_ref, wa_ref, wb_ref, a_ref, b_ref):
        xv = x_ref[...]
        a_ref[...] = jnp.dot(xv, wa_ref[...], preferred_element_type=jnp.float32)
        b_ref[...] = jnp.dot(xv, wb_ref[...], preferred_element_type=jnp.float32)

    return pl.pallas_call(
        body,
        out_shape=[
            jax.ShapeDtypeStruct((_N, _H), jnp.float32),
            jax.ShapeDtypeStruct((_N, _H), jnp.float32),
        ],
    )(x, w1a, w1b)


def _sc_gather(a, b, eni):
    mesh = plsc.VectorSubcoreMesh(core_axis_name="c", subcore_axis_name="s")

    nwin = _EW // _GW
    grows = _GW // 4

    nb = 2

    @functools.partial(
        pl.kernel,
        out_type=jax.ShapeDtypeStruct((_E4, 4 * _H), jnp.float32),
        mesh=mesh,
        compiler_params=pltpu.CompilerParams(use_tc_tiling_on_sc=False),
        scratch_types=(
            [pltpu.VMEM((_EW,), jnp.int32)] * 2
            + [pltpu.VMEM((_GW, _H), jnp.float32)] * (2 * nb)
            + [pltpu.VMEM((grows, 4 * _H), jnp.float32)] * nb
            + [pltpu.SemaphoreType.DMA] * (3 * nb)
        ),
    )
    def k(a_hbm, b_hbm, eni_hbm, g_hbm, i0, i1, *scr):
        abufs = scr[0:nb]
        bbufs = scr[nb:2 * nb]
        obufs = scr[2 * nb:3 * nb]
        sas = scr[3 * nb:4 * nb]
        sbs = scr[4 * nb:5 * nb]
        wss = scr[5 * nb:6 * nb]
        wid = lax.axis_index("s") * 2 + lax.axis_index("c")
        base = wid * _EW
        base4 = wid * (_EW // 4)
        pltpu.sync_copy(eni_hbm.at[0, pl.ds(base, _EW)], i0)
        pltpu.sync_copy(eni_hbm.at[1, pl.ds(base, _EW)], i1)

        def issue(w, j):
            off = w * _GW
            pltpu.async_copy(a_hbm.at[i0.at[pl.ds(off, _GW)]], abufs[j], sas[j])
            pltpu.async_copy(b_hbm.at[i1.at[pl.ds(off, _GW)]], bbufs[j], sbs[j])

        def wait_gathers(j):
            pltpu.make_async_copy(
                a_hbm.at[i0.at[pl.ds(0, _GW)]], abufs[j], sas[j]).wait()
            pltpu.make_async_copy(
                b_hbm.at[i1.at[pl.ds(0, _GW)]], bbufs[j], sbs[j]).wait()

        def add_rows(j):
            abuf, bbuf, obuf = abufs[j], bbufs[j], obufs[j]

            @pl.loop(0, grows)
            def _(q):
                for s in range(4):
                    for h0 in (0, 16):
                        obuf[q, pl.ds(s * _H + h0, 16)] = (
                            abuf[4 * q + s, pl.ds(h0, 16)]
                            + bbuf[4 * q + s, pl.ds(h0, 16)]
                        )

        def wait_write(j):
            pltpu.make_async_copy(obufs[j], g_hbm.at[pl.ds(0, grows)], wss[j]).wait()

        for j in range(nb):
            issue(j, j)

        @pl.loop(0, nwin - 1, step=nb)
        def _(ci):
            for j in range(nb):
                wait_gathers(j)

                @pl.when(ci >= nb)
                def _():
                    wait_write(j)

                add_rows(j)
                pltpu.async_copy(
                    obufs[j], g_hbm.at[pl.ds(base4 + (ci + j) * grows, grows)],
                    wss[j])

                @pl.when(ci + nb + j < nwin)
                def _():
                    issue(ci + nb + j, j)

        wait_gathers(0)
        wait_write(0)
        add_rows(0)
        pltpu.async_copy(
            obufs[0], g_hbm.at[pl.ds(base4 + (nwin - 1) * grows, grows)], wss[0])
        for j in range(nb):
            wait_write(j)

    return k(a, b, eni)


def _tc_edge_mlp(g4, ef4, w1c4, b1e4, w2e4, b2e4):
    rows = 4000
    grid = (_E4 // rows,)

    def body(g_ref, e_ref, w1_ref, b1_ref, w2_ref, b2_ref, o_ref):
        h = (
            g_ref[...]
            + jnp.dot(e_ref[...], w1_ref[...], preferred_element_type=jnp.float32)
            + b1_ref[...]
        )
        h = h * jax.nn.sigmoid(h)
        m = jnp.dot(h, w2_ref[...], preferred_element_type=jnp.float32) + b2_ref[...]
        o_ref[...] = m * jax.nn.sigmoid(m)

    return pl.pallas_call(
        body,
        grid=grid,
        in_specs=[
            pl.BlockSpec((rows, _FOLD * _H), lambda i: (i, 0)),
            pl.BlockSpec((rows, _FOLD * _FE), lambda i: (i, 0)),
            pl.BlockSpec((_FOLD * _FE, _FOLD * _H), lambda i: (0, 0)),
            pl.BlockSpec((1, _FOLD * _H), lambda i: (0, 0)),
            pl.BlockSpec((_FOLD * _H, _FOLD * _MSG), lambda i: (0, 0)),
            pl.BlockSpec((1, _FOLD * _MSG), lambda i: (0, 0)),
        ],
        out_specs=pl.BlockSpec((rows, _FOLD * _MSG), lambda i: (i, 0)),
        out_shape=jax.ShapeDtypeStruct((_E4, _FOLD * _MSG), jnp.float32),
    )(g4, ef4, w1c4, b1e4, w2e4, b2e4)


def _sc_scatter(eni, m_flat):
    mesh = plsc.VectorSubcoreMesh(core_axis_name="c", subcore_axis_name="s")

    nchunks = _EW // _CHUNK

    @functools.partial(
        pl.kernel,
        out_type=jax.ShapeDtypeStruct((_MSG * _NW, _N), jnp.float32),
        mesh=mesh,
        compiler_params=pltpu.CompilerParams(
            use_tc_tiling_on_sc=False, needs_layout_passes=False),
        scratch_types=[
            pltpu.VMEM((_MSG, _N), jnp.float32),
            pltpu.VMEM((_CHUNK,), jnp.int32),
            pltpu.VMEM((_CHUNK,), jnp.int32),
            pltpu.VMEM((_CHUNK * _MSG,), jnp.float32),
            pltpu.VMEM((_CHUNK * _MSG,), jnp.float32),
            pltpu.SemaphoreType.DMA,
            pltpu.SemaphoreType.DMA,
        ],
    )
    def k(eni_hbm, m_hbm, o_hbm, acc, idx0, idx1, mb0, mb1, s0, s1):
        wid = lax.axis_index("s") * 2 + lax.axis_index("c")
        idxs, mbs, sems = (idx0, idx1), (mb0, mb1), (s0, s1)

        def issue(ci, j):
            base = wid * _EW + ci * _CHUNK
            pltpu.async_copy(eni_hbm.at[0, pl.ds(base, _CHUNK)], idxs[j], sems[j])
            pltpu.async_copy(
                m_hbm.at[pl.ds(base * _MSG, _CHUNK * _MSG)], mbs[j], sems[j])

        def wait_in(j):
            pltpu.make_async_copy(
                eni_hbm.at[0, pl.ds(0, _CHUNK)], idxs[j], sems[j]).wait()
            pltpu.make_async_copy(
                m_hbm.at[pl.ds(0, _CHUNK * _MSG)], mbs[j], sems[j]).wait()

        issue(0, 0)
        issue(1, 1)

        @pl.loop(0, _N, step=80)
        def _(j):
            z = jnp.zeros((16,), jnp.float32)
            for r in range(4):
                for c in range(5):
                    acc[r, pl.ds(j + 16 * c, 16)] = z

        lanes = lax.broadcasted_iota(jnp.int32, (16,), 0)
        e_rep = lanes >> 2
        k_rep = lanes & 3

        @pl.loop(0, nchunks, step=2)
        def _(ci):
            for j in range(2):
                wait_in(j)

                @pl.loop(0, _CHUNK, step=4)
                def _(e0):
                    nvec = plsc.load_gather(idxs[j], [e0 + e_rep])
                    vals = mbs[j][pl.ds(e0 * _MSG, 16)]
                    plsc.addupdate_scatter(acc, [k_rep, nvec], vals, mask=e_rep == 0)
                    plsc.addupdate_scatter(acc, [k_rep, nvec], vals, mask=e_rep == 1)
                    plsc.addupdate_scatter(acc, [k_rep, nvec], vals, mask=e_rep == 2)
                    plsc.addupdate_scatter(acc, [k_rep, nvec], vals, mask=e_rep == 3)

                @pl.when(ci + 2 + j < nchunks)
                def _():
                    issue(ci + 2 + j, j)

        pltpu.sync_copy(acc, o_hbm.at[pl.ds(_MSG * wid, _MSG)])

    return k(eni, m_flat)


def _tc_node_mlp(x, pt, w1nx, w1nmt, b1n, w2n, b2n):
    def body(x_ref, p_ref, wx_ref, wm_ref, b1_ref, w2_ref, b2_ref, o_ref):
        h = (
            jnp.dot(x_ref[...], wx_ref[...], preferred_element_type=jnp.float32)
            + lax.dot_general(
                p_ref[...], wm_ref[...], (((0,), (0,)), ((), ())),
                preferred_element_type=jnp.float32,
            )
            + b1_ref[...]
        )
        h = h * jax.nn.sigmoid(h)
        o = jnp.dot(h, w2_ref[...], preferred_element_type=jnp.float32) + b2_ref[...]
        o_ref[...] = o * jax.nn.sigmoid(o)

    return pl.pallas_call(
        body,
        out_shape=jax.ShapeDtypeStruct((_N, _F), jnp.float32),
    )(x, pt, w1nx, w1nmt, b1n, w2n, b2n)


def kernel(node_features, edge_node_indices, edge_features,
           W1e, b1e, W2e, b2e, W1n, b1n, W2n, b2n):
    w1a, w1b, w1c = W1e[:_F], W1e[_F:2 * _F], W1e[2 * _F:]

    a, b = _tc_precompute(node_features, w1a, w1b)
    g4 = _sc_gather(a, b, edge_node_indices)

    eye = jnp.eye(_FOLD, dtype=jnp.float32)
    w1c4 = jnp.kron(eye, w1c)
    b1e4 = jnp.tile(b1e, _FOLD).reshape(1, -1)
    w2e4 = jnp.kron(eye, W2e)
    b2e4 = jnp.tile(b2e, _FOLD).reshape(1, -1)
    ef4 = edge_features.reshape(_E4, _FOLD * _FE)
    m4 = _tc_edge_mlp(g4, ef4, w1c4, b1e4, w2e4, b2e4)

    pt = _sc_scatter(edge_node_indices, m4.reshape(-1))

    w1nx, w1nm = W1n[:_F], W1n[_F:]
    w1nmt = jnp.tile(w1nm, (_NW, 1))
    return _tc_node_mlp(node_features, pt, w1nx, w1nmt,
                        b1n.reshape(1, -1), W2n, b2n.reshape(1, -1))


# speedup vs baseline: 1.0078x; 1.0078x over previous
import functools

import jax
import jax.numpy as jnp
from jax import lax
from jax.experimental import pallas as pl
from jax.experimental.pallas import tpu as pltpu
from jax.experimental.pallas import tpu_sc as plsc

_N = 10000
_E = 320000
_F = 128
_FE = 16
_H = 32
_MSG = 4

_NW = 32
_EW = _E // _NW
_GW = 80
_FOLD = 4
_E4 = _E // _FOLD
_CHUNK = 1000


def _tc_precompute(x, w1a, w1b):
    def body(x_ref, wa_ref, wb_ref, a_ref, b_ref):
        xv = x_ref[...]
        a_ref[...] = jnp.dot(xv, wa_ref[...], preferred_element_type=jnp.float32)
        b_ref[...] = jnp.dot(xv, wb_ref[...], preferred_element_type=jnp.float32)

    return pl.pallas_call(
        body,
        out_shape=[
            jax.ShapeDtypeStruct((_N, _H), jnp.float32),
            jax.ShapeDtypeStruct((_N, _H), jnp.float32),
        ],
    )(x, w1a, w1b)


def _sc_gather(a, b, eni):
    mesh = plsc.VectorSubcoreMesh(core_axis_name="c", subcore_axis_name="s")

    nwin = _EW // _GW
    grows = _GW // 4

    nb = 2

    @functools.partial(
        pl.kernel,
        out_type=jax.ShapeDtypeStruct((_E4, 4 * _H), jnp.float32),
        mesh=mesh,
        compiler_params=pltpu.CompilerParams(use_tc_tiling_on_sc=False),
        scratch_types=(
            [pltpu.VMEM((_EW,), jnp.int32)] * 2
            + [pltpu.VMEM((_GW, _H), jnp.float32)] * (2 * nb)
            + [pltpu.VMEM((grows, 4 * _H), jnp.float32)] * nb
            + [pltpu.SemaphoreType.DMA] * (3 * nb)
        ),
    )
    def k(a_hbm, b_hbm, eni_hbm, g_hbm, i0, i1, *scr):
        abufs = scr[0:nb]
        bbufs = scr[nb:2 * nb]
        obufs = scr[2 * nb:3 * nb]
        sas = scr[3 * nb:4 * nb]
        sbs = scr[4 * nb:5 * nb]
        wss = scr[5 * nb:6 * nb]
        wid = lax.axis_index("s") * 2 + lax.axis_index("c")
        base = wid * _EW
        base4 = wid * (_EW // 4)
        pltpu.sync_copy(eni_hbm.at[0, pl.ds(base, _EW)], i0)
        pltpu.sync_copy(eni_hbm.at[1, pl.ds(base, _EW)], i1)

        def issue(w, j):
            off = w * _GW
            pltpu.async_copy(a_hbm.at[i0.at[pl.ds(off, _GW)]], abufs[j], sas[j])
            pltpu.async_copy(b_hbm.at[i1.at[pl.ds(off, _GW)]], bbufs[j], sbs[j])

        def wait_gathers(j):
            pltpu.make_async_copy(
                a_hbm.at[i0.at[pl.ds(0, _GW)]], abufs[j], sas[j]).wait()
            pltpu.make_async_copy(
                b_hbm.at[i1.at[pl.ds(0, _GW)]], bbufs[j], sbs[j]).wait()

        def add_rows(j):
            abuf, bbuf, obuf = abufs[j], bbufs[j], obufs[j]

            @pl.loop(0, grows)
            def _(q):
                for s in range(4):
                    for h0 in (0, 16):
                        obuf[q, pl.ds(s * _H + h0, 16)] = (
                            abuf[4 * q + s, pl.ds(h0, 16)]
                            + bbuf[4 * q + s, pl.ds(h0, 16)]
                        )

        def wait_write(j):
            pltpu.make_async_copy(obufs[j], g_hbm.at[pl.ds(0, grows)], wss[j]).wait()

        for j in range(nb):
            issue(j, j)

        @pl.loop(0, nwin - 1, step=nb)
        def _(ci):
            for j in range(nb):
                wait_gathers(j)

                @pl.when(ci >= nb)
                def _():
                    wait_write(j)

                add_rows(j)
                pltpu.async_copy(
                    obufs[j], g_hbm.at[pl.ds(base4 + (ci + j) * grows, grows)],
                    wss[j])

                @pl.when(ci + nb + j < nwin)
                def _():
                    issue(ci + nb + j, j)

        wait_gathers(0)
        wait_write(0)
        add_rows(0)
        pltpu.async_copy(
            obufs[0], g_hbm.at[pl.ds(base4 + (nwin - 1) * grows, grows)], wss[0])
        for j in range(nb):
            wait_write(j)

    return k(a, b, eni)


def _tc_edge_mlp(g4, ef4, w1c4, b1e4, w2e4, b2e4):
    rows = 8000
    grid = (_E4 // rows,)

    def body(g_ref, e_ref, w1_ref, b1_ref, w2_ref, b2_ref, o_ref):
        h = (
            g_ref[...]
            + jnp.dot(e_ref[...], w1_ref[...], preferred_element_type=jnp.float32)
            + b1_ref[...]
        )
        h = h * jax.nn.sigmoid(h)
        m = jnp.dot(h, w2_ref[...], preferred_element_type=jnp.float32) + b2_ref[...]
        o_ref[...] = m * jax.nn.sigmoid(m)

    return pl.pallas_call(
        body,
        grid=grid,
        in_specs=[
            pl.BlockSpec((rows, _FOLD * _H), lambda i: (i, 0)),
            pl.BlockSpec((rows, _FOLD * _FE), lambda i: (i, 0)),
            pl.BlockSpec((_FOLD * _FE, _FOLD * _H), lambda i: (0, 0)),
            pl.BlockSpec((1, _FOLD * _H), lambda i: (0, 0)),
            pl.BlockSpec((_FOLD * _H, _FOLD * _MSG), lambda i: (0, 0)),
            pl.BlockSpec((1, _FOLD * _MSG), lambda i: (0, 0)),
        ],
        out_specs=pl.BlockSpec((rows, _FOLD * _MSG), lambda i: (i, 0)),
        out_shape=jax.ShapeDtypeStruct((_E4, _FOLD * _MSG), jnp.float32),
    )(g4, ef4, w1c4, b1e4, w2e4, b2e4)


def _sc_scatter(eni, m_flat):
    mesh = plsc.VectorSubcoreMesh(core_axis_name="c", subcore_axis_name="s")

    nchunks = _EW // _CHUNK

    @functools.partial(
        pl.kernel,
        out_type=jax.ShapeDtypeStruct((_MSG * _NW, _N), jnp.float32),
        mesh=mesh,
        compiler_params=pltpu.CompilerParams(
            use_tc_tiling_on_sc=False, needs_layout_passes=False),
        scratch_types=[
            pltpu.VMEM((_MSG, _N), jnp.float32),
            pltpu.VMEM((_CHUNK,), jnp.int32),
            pltpu.VMEM((_CHUNK,), jnp.int32),
            pltpu.VMEM((_CHUNK * _MSG,), jnp.float32),
            pltpu.VMEM((_CHUNK * _MSG,), jnp.float32),
            pltpu.SemaphoreType.DMA,
            pltpu.SemaphoreType.DMA,
        ],
    )
    def k(eni_hbm, m_hbm, o_hbm, acc, idx0, idx1, mb0, mb1, s0, s1):
        wid = lax.axis_index("s") * 2 + lax.axis_index("c")
        idxs, mbs, sems = (idx0, idx1), (mb0, mb1), (s0, s1)

        def issue(ci, j):
            base = wid * _EW + ci * _CHUNK
            pltpu.async_copy(eni_hbm.at[0, pl.ds(base, _CHUNK)], idxs[j], sems[j])
            pltpu.async_copy(
                m_hbm.at[pl.ds(base * _MSG, _CHUNK * _MSG)], mbs[j], sems[j])

        def wait_in(j):
            pltpu.make_async_copy(
                eni_hbm.at[0, pl.ds(0, _CHUNK)], idxs[j], sems[j]).wait()
            pltpu.make_async_copy(
                m_hbm.at[pl.ds(0, _CHUNK * _MSG)], mbs[j], sems[j]).wait()

        issue(0, 0)
        issue(1, 1)

        @pl.loop(0, _N, step=80)
        def _(j):
            z = jnp.zeros((16,), jnp.float32)
            for r in range(4):
                for c in range(5):
                    acc[r, pl.ds(j + 16 * c, 16)] = z

        lanes = lax.broadcasted_iota(jnp.int32, (16,), 0)
        e_rep = lanes >> 2
        k_rep = lanes & 3

        @pl.loop(0, nchunks, step=2)
        def _(ci):
            for j in range(2):
                wait_in(j)

                @pl.loop(0, _CHUNK, step=4)
                def _(e0):
                    nvec = plsc.load_gather(idxs[j], [e0 + e_rep])
                    vals = mbs[j][pl.ds(e0 * _MSG, 16)]
                    plsc.addupdate_scatter(acc, [k_rep, nvec], vals, mask=e_rep == 0)
                    plsc.addupdate_scatter(acc, [k_rep, nvec], vals, mask=e_rep == 1)
                    plsc.addupdate_scatter(acc, [k_rep, nvec], vals, mask=e_rep == 2)
                    plsc.addupdate_scatter(acc, [k_rep, nvec], vals, mask=e_rep == 3)

                @pl.when(ci + 2 + j < nchunks)
                def _():
                    issue(ci + 2 + j, j)

        pltpu.sync_copy(acc, o_hbm.at[pl.ds(_MSG * wid, _MSG)])

    return k(eni, m_flat)


def _tc_node_mlp(x, pt, w1nx, w1nmt, b1n, w2n, b2n):
    def body(x_ref, p_ref, wx_ref, wm_ref, b1_ref, w2_ref, b2_ref, o_ref):
        h = (
            jnp.dot(x_ref[...], wx_ref[...], preferred_element_type=jnp.float32)
            + lax.dot_general(
                p_ref[...], wm_ref[...], (((0,), (0,)), ((), ())),
                preferred_element_type=jnp.float32,
            )
            + b1_ref[...]
        )
        h = h * jax.nn.sigmoid(h)
        o = jnp.dot(h, w2_ref[...], preferred_element_type=jnp.float32) + b2_ref[...]
        o_ref[...] = o * jax.nn.sigmoid(o)

    return pl.pallas_call(
        body,
        out_shape=jax.ShapeDtypeStruct((_N, _F), jnp.float32),
    )(x, pt, w1nx, w1nmt, b1n, w2n, b2n)


def kernel(node_features, edge_node_indices, edge_features,
           W1e, b1e, W2e, b2e, W1n, b1n, W2n, b2n):
    w1a, w1b, w1c = W1e[:_F], W1e[_F:2 * _F], W1e[2 * _F:]

    a, b = _tc_precompute(node_features, w1a, w1b)
    g4 = _sc_gather(a, b, edge_node_indices)

    eye = jnp.eye(_FOLD, dtype=jnp.float32)
    w1c4 = jnp.kron(eye, w1c)
    b1e4 = jnp.tile(b1e, _FOLD).reshape(1, -1)
    w2e4 = jnp.kron(eye, W2e)
    b2e4 = jnp.tile(b2e, _FOLD).reshape(1, -1)
    ef4 = edge_features.reshape(_E4, _FOLD * _FE)
    m4 = _tc_edge_mlp(g4, ef4, w1c4, b1e4, w2e4, b2e4)

    pt = _sc_scatter(edge_node_indices, m4.reshape(-1))

    w1nx, w1nm = W1n[:_F], W1n[_F:]
    w1nmt = jnp.tile(w1nm, (_NW, 1))
    return _tc_node_mlp(node_features, pt, w1nx, w1nmt,
                        b1n.reshape(1, -1), W2n, b2n.reshape(1, -1))
